# Initial kernel scaffold; baseline (speedup 1.0000x reference)
import jax
import jax.numpy as jnp
from jax.experimental import pallas as pl


def kernel(charges, pair_dist, pair_first, pair_second, mol_index, n_molecules):
    raise NotImplementedError("write your pallas kernel here")


# speedup vs baseline: 362.3118x; 362.3118x over previous
import functools

import jax
import jax.numpy as jnp
from jax import lax
from jax.experimental import pallas as pl
from jax.experimental.pallas import tpu as pltpu
from jax.experimental.pallas import tpu_sc as plsc

N_ATOMS = 100000
N_PAIRS = 6400000
N_MOL = 1000
RADIUS = 5.0
ENERGY_CONV = 14.399645

NC = 2
NS = 16
L = 16
NW = NC * NS
PER_TILE = N_PAIRS // NW
BLK = 2000
NBLK = PER_TILE // BLK
ACC_W = 1024

_C = (7.1998224, -17.764847, 14.610941, -4.806452,
      0.84608644, -0.091290094, 0.005737937)


def _sc_body(tab_hbm, pf_hbm, ps_hbm, pd_hbm, out_hbm,
             tab_v, i1_v, i2_v, d_v, acc_v, row_v):
    wid = lax.axis_index("s") * NC + lax.axis_index("c")
    base = wid * PER_TILE

    pltpu.sync_copy(tab_hbm, tab_v)

    def zero(j, _):
        acc_v[pl.ds(j * L, L)] = jnp.zeros((L,), jnp.float32)
        return 0
    lax.fori_loop(0, (L * ACC_W) // L, zero, 0)

    lane_base = lax.iota(jnp.int32, L) * ACC_W
    inv_r2 = jnp.float32(1.0 / (RADIUS * RADIUS))

    def block(b, _):
        off = base + b * BLK
        pltpu.sync_copy(pf_hbm.at[pl.ds(off, BLK)], i1_v)
        pltpu.sync_copy(ps_hbm.at[pl.ds(off, BLK)], i2_v)
        pltpu.sync_copy(pd_hbm.at[pl.ds(off, BLK)], d_v)

        def inner(i, _):
            i1 = i1_v[pl.ds(i * L, L)]
            i2 = i2_v[pl.ds(i * L, L)]
            d = d_v[pl.ds(i * L, L)]
            t1 = plsc.load_gather(tab_v, [i1])
            t2 = plsc.load_gather(tab_v, [i2])
            m = jnp.bitwise_and(t1, jnp.int32(1023))
            q1 = plsc.bitcast(jnp.bitwise_and(t1, jnp.int32(-1024)), jnp.float32)
            q2 = plsc.bitcast(jnp.bitwise_and(t2, jnp.int32(-1024)), jnp.float32)
            u = d * d * inv_r2
            p = jnp.float32(_C[6])
            p = p * u + jnp.float32(_C[5])
            p = p * u + jnp.float32(_C[4])
            p = p * u + jnp.float32(_C[3])
            p = p * u + jnp.float32(_C[2])
            p = p * u + jnp.float32(_C[1])
            p = p * u + jnp.float32(_C[0])
            e = q1 * q2 * (p / d)
            e = jnp.where(d < jnp.float32(RADIUS), e, jnp.float32(0.0))
            plsc.addupdate_scatter(acc_v, [lane_base + m], e)
            return 0

        lax.fori_loop(0, BLK // L, inner, 0)
        return 0

    lax.fori_loop(0, NBLK, block, 0)

    def fold(j, _):
        s = acc_v[pl.ds(j * L, L)]
        for r in range(1, L):
            s = s + acc_v[pl.ds(r * ACC_W + j * L, L)]
        row_v[pl.ds(j * L, L)] = s
        return 0
    lax.fori_loop(0, ACC_W // L, fold, 0)

    pltpu.sync_copy(row_v, out_hbm.at[wid])


_sc_kernel = functools.partial(
    pl.kernel,
    out_type=jax.ShapeDtypeStruct((NW, ACC_W), jnp.float32),
    mesh=plsc.VectorSubcoreMesh(
        core_axis_name="c", subcore_axis_name="s",
        num_cores=NC, num_subcores=NS),
    compiler_params=pltpu.CompilerParams(needs_layout_passes=False),
    scratch_types=[
        pltpu.VMEM((N_ATOMS,), jnp.int32),
        pltpu.VMEM((BLK,), jnp.int32),
        pltpu.VMEM((BLK,), jnp.int32),
        pltpu.VMEM((BLK,), jnp.float32),
        pltpu.VMEM((L * ACC_W,), jnp.float32),
        pltpu.VMEM((ACC_W,), jnp.float32),
    ],
)(_sc_body)


def _tc_reduce_body(p_ref, o_ref):
    o_ref[...] = jnp.sum(p_ref[...], axis=0, keepdims=True)


_tc_reduce = pl.pallas_call(
    _tc_reduce_body,
    out_shape=jax.ShapeDtypeStruct((1, ACC_W), jnp.float32),
)


def kernel(charges, pair_dist, pair_first, pair_second, mol_index, n_molecules):
    q = charges.reshape(N_ATOMS)
    bits = lax.bitcast_convert_type(q, jnp.int32)
    bits = jnp.bitwise_and(bits + jnp.int32(512), jnp.int32(-1024))
    packed = jnp.bitwise_or(bits, mol_index)
    partials = _sc_kernel(packed, pair_first, pair_second, pair_dist)
    row = _tc_reduce(partials)
    return row[0, :N_MOL].reshape(N_MOL, 1)


# speedup vs baseline: 560.4318x; 1.5468x over previous
import functools

import jax
import jax.numpy as jnp
from jax import lax
from jax.experimental import pallas as pl
from jax.experimental.pallas import tpu as pltpu
from jax.experimental.pallas import tpu_sc as plsc

N_ATOMS = 100000
N_PAIRS = 6400000
N_MOL = 1000
RADIUS = 5.0
ENERGY_CONV = 14.399645

NC = 2
NS = 16
L = 16
NW = NC * NS
PER_TILE = N_PAIRS // NW
BLK = 2000
NBLK = PER_TILE // BLK
VREGS = BLK // L
UNROLL = 5
ACC_W = 1024

_C = (7.1998224, -17.764847, 14.610941, -4.806452,
      0.84608644, -0.091290094, 0.005737937)


def _sc_body(tab_hbm, pf_hbm, ps_hbm, pd_hbm, out_hbm,
             tab_v, i1_v, i2_v, d_v, acc_v, row_v, sems):
    wid = lax.axis_index("s") * NC + lax.axis_index("c")
    base = wid * PER_TILE

    pltpu.sync_copy(tab_hbm, tab_v)

    def zero(j, _):
        acc_v[pl.ds(j * L, L)] = jnp.zeros((L,), jnp.float32)
        return 0
    lax.fori_loop(0, (L * ACC_W) // L, zero, 0)

    lane_base = lax.iota(jnp.int32, L) * ACC_W
    inv_r2 = jnp.float32(1.0 / (RADIUS * RADIUS))

    def copies(b, sel):
        off = base + b * BLK
        dst = pl.ds(sel * BLK, BLK)
        return (
            pltpu.make_async_copy(pf_hbm.at[pl.ds(off, BLK)], i1_v.at[dst],
                                  sems.at[sel, 0]),
            pltpu.make_async_copy(ps_hbm.at[pl.ds(off, BLK)], i2_v.at[dst],
                                  sems.at[sel, 1]),
            pltpu.make_async_copy(pd_hbm.at[pl.ds(off, BLK)], d_v.at[dst],
                                  sems.at[sel, 2]),
        )

    for c in copies(0, 0):
        c.start()

    def block(b, _):
        sel = jnp.bitwise_and(b, 1)

        @pl.when(b < NBLK - 1)
        def _():
            for c in copies(b + 1, 1 - sel):
                c.start()

        for c in copies(b, sel):
            c.wait()

        vbase = sel * BLK

        def inner(i, _):
            for k in range(UNROLL):
                o = vbase + (i * UNROLL + k) * L
                i1 = i1_v[pl.ds(o, L)]
                i2 = i2_v[pl.ds(o, L)]
                d = d_v[pl.ds(o, L)]
                t1 = plsc.load_gather(tab_v, [i1])
                t2 = plsc.load_gather(tab_v, [i2])
                m = jnp.bitwise_and(t1, jnp.int32(1023))
                q1 = plsc.bitcast(jnp.bitwise_and(t1, jnp.int32(-1024)),
                                  jnp.float32)
                q2 = plsc.bitcast(jnp.bitwise_and(t2, jnp.int32(-1024)),
                                  jnp.float32)
                u = d * d * inv_r2
                p = jnp.float32(_C[6])
                p = p * u + jnp.float32(_C[5])
                p = p * u + jnp.float32(_C[4])
                p = p * u + jnp.float32(_C[3])
                p = p * u + jnp.float32(_C[2])
                p = p * u + jnp.float32(_C[1])
                p = p * u + jnp.float32(_C[0])
                e = q1 * q2 * (p / d)
                e = jnp.where(d < jnp.float32(RADIUS), e, jnp.float32(0.0))
                plsc.addupdate_scatter(acc_v, [lane_base + m], e)
            return 0

        lax.fori_loop(0, VREGS // UNROLL, inner, 0)
        return 0

    lax.fori_loop(0, NBLK, block, 0)

    def fold(j, _):
        s = acc_v[pl.ds(j * L, L)]
        for r in range(1, L):
            s = s + acc_v[pl.ds(r * ACC_W + j * L, L)]
        row_v[pl.ds(j * L, L)] = s
        return 0
    lax.fori_loop(0, ACC_W // L, fold, 0)

    pltpu.sync_copy(row_v, out_hbm.at[wid])


_sc_kernel = functools.partial(
    pl.kernel,
    out_type=jax.ShapeDtypeStruct((NW, ACC_W), jnp.float32),
    mesh=plsc.VectorSubcoreMesh(
        core_axis_name="c", subcore_axis_name="s",
        num_cores=NC, num_subcores=NS),
    compiler_params=pltpu.CompilerParams(needs_layout_passes=False),
    scratch_types=[
        pltpu.VMEM((N_ATOMS,), jnp.int32),
        pltpu.VMEM((2 * BLK,), jnp.int32),
        pltpu.VMEM((2 * BLK,), jnp.int32),
        pltpu.VMEM((2 * BLK,), jnp.float32),
        pltpu.VMEM((L * ACC_W,), jnp.float32),
        pltpu.VMEM((ACC_W,), jnp.float32),
        pltpu.SemaphoreType.DMA((2, 3)),
    ],
)(_sc_body)


def _tc_reduce_body(p_ref, o_ref):
    o_ref[...] = jnp.sum(p_ref[...], axis=0, keepdims=True)


_tc_reduce = pl.pallas_call(
    _tc_reduce_body,
    out_shape=jax.ShapeDtypeStruct((1, ACC_W), jnp.float32),
)


def kernel(charges, pair_dist, pair_first, pair_second, mol_index, n_molecules):
    q = charges.reshape(N_ATOMS)
    bits = lax.bitcast_convert_type(q, jnp.int32)
    bits = jnp.bitwise_and(bits + jnp.int32(512), jnp.int32(-1024))
    packed = jnp.bitwise_or(bits, mol_index)
    partials = _sc_kernel(packed, pair_first, pair_second, pair_dist)
    row = _tc_reduce(partials)
    return row[0, :N_MOL].reshape(N_MOL, 1)


# speedup vs baseline: 1592.8460x; 2.8422x over previous
import functools

import jax
import jax.numpy as jnp
from jax import lax
from jax.experimental import pallas as pl
from jax.experimental.pallas import tpu as pltpu
from jax.experimental.pallas import tpu_sc as plsc

N_ATOMS = 100000
N_PAIRS = 6400000
N_MOL = 1000
RADIUS = 5.0
ENERGY_CONV = 14.399645

NC = 2
NS = 16
L = 16
NW = NC * NS
PER_TILE = N_PAIRS // NW
BLK = 2000
NBLK = PER_TILE // BLK
VREGS = BLK // L
UNROLL = 5
ACC_W = 1024

_C = (7.1998224, -17.764847, 14.610941, -4.806452,
      0.84608644, -0.091290094, 0.005737937)


def _sc_body(tab_hbm, pf_hbm, ps_hbm, pd_hbm, out_hbm,
             tab_v, i1_v, i2_v, d_v, acc_v, row_v, sems):
    wid = lax.axis_index("s") * NC + lax.axis_index("c")
    base = wid * PER_TILE

    pltpu.sync_copy(tab_hbm, tab_v)

    def zero(j, _):
        acc_v[pl.ds(j * L, L)] = jnp.zeros((L,), jnp.float32)
        return 0
    lax.fori_loop(0, (L * ACC_W) // L, zero, 0)

    lane_base = lax.iota(jnp.int32, L) * ACC_W
    inv_r2 = jnp.float32(1.0 / (RADIUS * RADIUS))

    def copies(b, sel):
        off = base + b * BLK
        dst = pl.ds(sel * BLK, BLK)
        return (
            pltpu.make_async_copy(pf_hbm.at[pl.ds(off, BLK)], i1_v.at[dst],
                                  sems.at[sel, 0]),
            pltpu.make_async_copy(ps_hbm.at[pl.ds(off, BLK)], i2_v.at[dst],
                                  sems.at[sel, 1]),
            pltpu.make_async_copy(pd_hbm.at[pl.ds(off, BLK)], d_v.at[dst],
                                  sems.at[sel, 2]),
        )

    for c in copies(0, 0):
        c.start()

    def block(b, _):
        sel = jnp.bitwise_and(b, 1)

        @pl.when(b < NBLK - 1)
        def _():
            for c in copies(b + 1, 1 - sel):
                c.start()

        for c in copies(b, sel):
            c.wait()

        vbase = sel * BLK

        @plsc.parallel_loop(0, VREGS, step=1, unroll=UNROLL)
        def inner(i):
            o = vbase + i * L
            i1 = i1_v[pl.ds(o, L)]
            i2 = i2_v[pl.ds(o, L)]
            d = d_v[pl.ds(o, L)]
            t1 = plsc.load_gather(tab_v, [i1])
            t2 = plsc.load_gather(tab_v, [i2])
            m = jnp.bitwise_and(t1, jnp.int32(1023))
            q1 = plsc.bitcast(jnp.bitwise_and(t1, jnp.int32(-1024)),
                              jnp.float32)
            q2 = plsc.bitcast(jnp.bitwise_and(t2, jnp.int32(-1024)),
                              jnp.float32)
            u = d * d * inv_r2
            p = jnp.float32(_C[6])
            p = p * u + jnp.float32(_C[5])
            p = p * u + jnp.float32(_C[4])
            p = p * u + jnp.float32(_C[3])
            p = p * u + jnp.float32(_C[2])
            p = p * u + jnp.float32(_C[1])
            p = p * u + jnp.float32(_C[0])
            e = q1 * q2 * (p / d)
            e = jnp.where(d < jnp.float32(RADIUS), e, jnp.float32(0.0))
            plsc.addupdate_scatter(acc_v, [lane_base + m], e)

        return 0

    lax.fori_loop(0, NBLK, block, 0)

    def fold(j, _):
        s = acc_v[pl.ds(j * L, L)]
        for r in range(1, L):
            s = s + acc_v[pl.ds(r * ACC_W + j * L, L)]
        row_v[pl.ds(j * L, L)] = s
        return 0
    lax.fori_loop(0, ACC_W // L, fold, 0)

    pltpu.sync_copy(row_v, out_hbm.at[wid])


_sc_kernel = functools.partial(
    pl.kernel,
    out_type=jax.ShapeDtypeStruct((NW, ACC_W), jnp.float32),
    mesh=plsc.VectorSubcoreMesh(
        core_axis_name="c", subcore_axis_name="s",
        num_cores=NC, num_subcores=NS),
    compiler_params=pltpu.CompilerParams(needs_layout_passes=False),
    scratch_types=[
        pltpu.VMEM((N_ATOMS,), jnp.int32),
        pltpu.VMEM((2 * BLK,), jnp.int32),
        pltpu.VMEM((2 * BLK,), jnp.int32),
        pltpu.VMEM((2 * BLK,), jnp.float32),
        pltpu.VMEM((L * ACC_W,), jnp.float32),
        pltpu.VMEM((ACC_W,), jnp.float32),
        pltpu.SemaphoreType.DMA((2, 3)),
    ],
)(_sc_body)


def _tc_reduce_body(p_ref, o_ref):
    o_ref[...] = jnp.sum(p_ref[...], axis=0, keepdims=True)


_tc_reduce = pl.pallas_call(
    _tc_reduce_body,
    out_shape=jax.ShapeDtypeStruct((1, ACC_W), jnp.float32),
)


def kernel(charges, pair_dist, pair_first, pair_second, mol_index, n_molecules):
    q = charges.reshape(N_ATOMS)
    bits = lax.bitcast_convert_type(q, jnp.int32)
    bits = jnp.bitwise_and(bits + jnp.int32(512), jnp.int32(-1024))
    packed = jnp.bitwise_or(bits, mol_index)
    partials = _sc_kernel(packed, pair_first, pair_second, pair_dist)
    row = _tc_reduce(partials)
    return row[0, :N_MOL].reshape(N_MOL, 1)


# speedup vs baseline: 1692.3165x; 1.0624x over previous
import functools

import jax
import jax.numpy as jnp
from jax import lax
from jax.experimental import pallas as pl
from jax.experimental.pallas import tpu as pltpu
from jax.experimental.pallas import tpu_sc as plsc

N_ATOMS = 100000
N_PAIRS = 6400000
N_MOL = 1000
RADIUS = 5.0
ENERGY_CONV = 14.399645

NC = 2
NS = 16
L = 16
NW = NC * NS
PER_TILE = N_PAIRS // NW
BLK = 2000
NBLK = PER_TILE // BLK
VREGS = BLK // L
UNROLL = 5
ACC_W = 1024

_C = (7.1992545, -0.70983636, 0.023198491, -0.00029170883, 1.5502035e-06)


def _sc_body(tab_hbm, pf_hbm, ps_hbm, pd_hbm, out_hbm,
             tab_v, i1_v, i2_v, d_v, acc_v, row_v, sems):
    wid = lax.axis_index("s") * NC + lax.axis_index("c")
    base = wid * PER_TILE

    pltpu.sync_copy(tab_hbm, tab_v)

    def zero(j, _):
        acc_v[pl.ds(j * L, L)] = jnp.zeros((L,), jnp.float32)
        return 0
    lax.fori_loop(0, (L * ACC_W) // L, zero, 0)

    lane_base = lax.iota(jnp.int32, L) * ACC_W

    def copies(b, sel):
        off = base + b * BLK
        dst = pl.ds(sel * BLK, BLK)
        return (
            pltpu.make_async_copy(pf_hbm.at[pl.ds(off, BLK)], i1_v.at[dst],
                                  sems.at[sel, 0]),
            pltpu.make_async_copy(ps_hbm.at[pl.ds(off, BLK)], i2_v.at[dst],
                                  sems.at[sel, 1]),
            pltpu.make_async_copy(pd_hbm.at[pl.ds(off, BLK)], d_v.at[dst],
                                  sems.at[sel, 2]),
        )

    for c in copies(0, 0):
        c.start()

    def block(b, _):
        sel = jnp.bitwise_and(b, 1)

        @pl.when(b < NBLK - 1)
        def _():
            for c in copies(b + 1, 1 - sel):
                c.start()

        for c in copies(b, sel):
            c.wait()

        vbase = sel * BLK

        @plsc.parallel_loop(0, VREGS, step=1, unroll=UNROLL)
        def inner(i):
            o = vbase + i * L
            i1 = i1_v[pl.ds(o, L)]
            i2 = i2_v[pl.ds(o, L)]
            d = d_v[pl.ds(o, L)]
            t1 = plsc.load_gather(tab_v, [i1])
            t2 = plsc.load_gather(tab_v, [i2])
            m = jnp.bitwise_and(t1, jnp.int32(1023))
            q1 = plsc.bitcast(jnp.bitwise_and(t1, jnp.int32(-1024)),
                              jnp.float32)
            q2 = plsc.bitcast(jnp.bitwise_and(t2, jnp.int32(-1024)),
                              jnp.float32)
            v = jnp.minimum(d * d, jnp.float32(RADIUS * RADIUS))
            p = jnp.float32(_C[4])
            p = p * v + jnp.float32(_C[3])
            p = p * v + jnp.float32(_C[2])
            p = p * v + jnp.float32(_C[1])
            p = p * v + jnp.float32(_C[0])
            e = q1 * q2 * (p / d)
            plsc.addupdate_scatter(acc_v, [lane_base + m], e)

        return 0

    lax.fori_loop(0, NBLK, block, 0)

    def fold(j, _):
        s = acc_v[pl.ds(j * L, L)]
        for r in range(1, L):
            s = s + acc_v[pl.ds(r * ACC_W + j * L, L)]
        row_v[pl.ds(j * L, L)] = s
        return 0
    lax.fori_loop(0, ACC_W // L, fold, 0)

    pltpu.sync_copy(row_v, out_hbm.at[wid])


_sc_kernel = functools.partial(
    pl.kernel,
    out_type=jax.ShapeDtypeStruct((NW, ACC_W), jnp.float32),
    mesh=plsc.VectorSubcoreMesh(
        core_axis_name="c", subcore_axis_name="s",
        num_cores=NC, num_subcores=NS),
    compiler_params=pltpu.CompilerParams(needs_layout_passes=False),
    scratch_types=[
        pltpu.VMEM((N_ATOMS,), jnp.int32),
        pltpu.VMEM((2 * BLK,), jnp.int32),
        pltpu.VMEM((2 * BLK,), jnp.int32),
        pltpu.VMEM((2 * BLK,), jnp.float32),
        pltpu.VMEM((L * ACC_W,), jnp.float32),
        pltpu.VMEM((ACC_W,), jnp.float32),
        pltpu.SemaphoreType.DMA((2, 3)),
    ],
)(_sc_body)


def _tc_reduce_body(p_ref, o_ref):
    o_ref[...] = jnp.sum(p_ref[...], axis=0, keepdims=True)


_tc_reduce = pl.pallas_call(
    _tc_reduce_body,
    out_shape=jax.ShapeDtypeStruct((1, ACC_W), jnp.float32),
)


def kernel(charges, pair_dist, pair_first, pair_second, mol_index, n_molecules):
    q = charges.reshape(N_ATOMS)
    bits = lax.bitcast_convert_type(q, jnp.int32)
    bits = jnp.bitwise_and(bits + jnp.int32(512), jnp.int32(-1024))
    packed = jnp.bitwise_or(bits, mol_index)
    partials = _sc_kernel(packed, pair_first, pair_second, pair_dist)
    row = _tc_reduce(partials)
    return row[0, :N_MOL].reshape(N_MOL, 1)


# speedup vs baseline: 1706.5553x; 1.0084x over previous
import functools

import jax
import jax.numpy as jnp
from jax import lax
from jax.experimental import pallas as pl
from jax.experimental.pallas import tpu as pltpu
from jax.experimental.pallas import tpu_sc as plsc

N_ATOMS = 100000
N_PAIRS = 6400000
N_MOL = 1000
RADIUS = 5.0
ENERGY_CONV = 14.399645

NC = 2
NS = 16
L = 16
NW = NC * NS
PER_TILE = N_PAIRS // NW
BLK = 2000
NBLK = PER_TILE // BLK
VREGS = BLK // L
UNROLL = 8
ACC_W = 1024

_C = (7.1992545, -0.70983636, 0.023198491, -0.00029170883, 1.5502035e-06)


def _sc_body(tab_hbm, pf_hbm, ps_hbm, pd_hbm, out_hbm,
             tab_v, i1_v, i2_v, d_v, acc_v, row_v, sems, tab_sem):
    wid = lax.axis_index("s") * NC + lax.axis_index("c")
    base = wid * PER_TILE

    tab_cp = pltpu.make_async_copy(tab_hbm, tab_v, tab_sem)
    tab_cp.start()

    @plsc.parallel_loop(0, (L * ACC_W) // L, step=1, unroll=8)
    def zero(j):
        acc_v[pl.ds(j * L, L)] = jnp.zeros((L,), jnp.float32)

    tab_cp.wait()

    lane_base = lax.iota(jnp.int32, L) * ACC_W

    def copies(b, sel):
        off = base + b * BLK
        dst = pl.ds(sel * BLK, BLK)
        return (
            pltpu.make_async_copy(pf_hbm.at[pl.ds(off, BLK)], i1_v.at[dst],
                                  sems.at[sel, 0]),
            pltpu.make_async_copy(ps_hbm.at[pl.ds(off, BLK)], i2_v.at[dst],
                                  sems.at[sel, 1]),
            pltpu.make_async_copy(pd_hbm.at[pl.ds(off, BLK)], d_v.at[dst],
                                  sems.at[sel, 2]),
        )

    for c in copies(0, 0):
        c.start()

    def block(b, _):
        sel = jnp.bitwise_and(b, 1)

        @pl.when(b < NBLK - 1)
        def _():
            for c in copies(b + 1, 1 - sel):
                c.start()

        for c in copies(b, sel):
            c.wait()

        vbase = sel * BLK

        @plsc.parallel_loop(0, VREGS, step=1, unroll=UNROLL)
        def inner(i):
            o = vbase + i * L
            i1 = i1_v[pl.ds(o, L)]
            i2 = i2_v[pl.ds(o, L)]
            d = d_v[pl.ds(o, L)]
            t1 = plsc.load_gather(tab_v, [i1])
            t2 = plsc.load_gather(tab_v, [i2])
            m = jnp.bitwise_and(t1, jnp.int32(1023))
            q1 = plsc.bitcast(jnp.bitwise_and(t1, jnp.int32(-1024)),
                              jnp.float32)
            q2 = plsc.bitcast(jnp.bitwise_and(t2, jnp.int32(-1024)),
                              jnp.float32)
            v = jnp.minimum(d * d, jnp.float32(RADIUS * RADIUS))
            p = jnp.float32(_C[4])
            p = p * v + jnp.float32(_C[3])
            p = p * v + jnp.float32(_C[2])
            p = p * v + jnp.float32(_C[1])
            p = p * v + jnp.float32(_C[0])
            e = q1 * q2 * (p / d)
            plsc.addupdate_scatter(acc_v, [lane_base + m], e)

        return 0

    lax.fori_loop(0, NBLK, block, 0)

    def fold(j, _):
        s = acc_v[pl.ds(j * L, L)]
        for r in range(1, L):
            s = s + acc_v[pl.ds(r * ACC_W + j * L, L)]
        row_v[pl.ds(j * L, L)] = s
        return 0
    lax.fori_loop(0, ACC_W // L, fold, 0)

    pltpu.sync_copy(row_v, out_hbm.at[wid])


_sc_kernel = functools.partial(
    pl.kernel,
    out_type=jax.ShapeDtypeStruct((NW, ACC_W), jnp.float32),
    mesh=plsc.VectorSubcoreMesh(
        core_axis_name="c", subcore_axis_name="s",
        num_cores=NC, num_subcores=NS),
    compiler_params=pltpu.CompilerParams(needs_layout_passes=False),
    scratch_types=[
        pltpu.VMEM((N_ATOMS,), jnp.int32),
        pltpu.VMEM((2 * BLK,), jnp.int32),
        pltpu.VMEM((2 * BLK,), jnp.int32),
        pltpu.VMEM((2 * BLK,), jnp.float32),
        pltpu.VMEM((L * ACC_W,), jnp.float32),
        pltpu.VMEM((ACC_W,), jnp.float32),
        pltpu.SemaphoreType.DMA((2, 3)),
        pltpu.SemaphoreType.DMA,
    ],
)(_sc_body)


def _tc_reduce_body(p_ref, o_ref):
    o_ref[...] = jnp.sum(p_ref[...], axis=0, keepdims=True)


_tc_reduce = pl.pallas_call(
    _tc_reduce_body,
    out_shape=jax.ShapeDtypeStruct((1, ACC_W), jnp.float32),
)


def kernel(charges, pair_dist, pair_first, pair_second, mol_index, n_molecules):
    q = charges.reshape(N_ATOMS)
    bits = lax.bitcast_convert_type(q, jnp.int32)
    bits = jnp.bitwise_and(bits + jnp.int32(512), jnp.int32(-1024))
    packed = jnp.bitwise_or(bits, mol_index)
    partials = _sc_kernel(packed, pair_first, pair_second, pair_dist)
    row = _tc_reduce(partials)
    return row[0, :N_MOL].reshape(N_MOL, 1)
